# baseline (speedup 1.0000x reference)
import functools

import jax
import jax.numpy as jnp
from jax import lax
from jax.experimental import pallas as pl
from jax.experimental.pallas import tpu as pltpu
from jax.experimental.pallas import tpu_sc as plsc

BATCH = 1024
WORDS = 200
COORDS = 64
TOKENS = 1000000
NUM_WORKERS = 32
W_PARTS = 4
B_COLS = 8
W_PER_TILE = WORDS // W_PARTS
BW = 128
STAGE_ROWS = 56
GPITCH = 128


TCH = 384
N_FULL = TOKENS // TCH
TAIL = TOKENS - N_FULL * TCH


def _pair_transpose_sc(table_ct, table_tail):
    mesh = plsc.VectorSubcoreMesh(core_axis_name="c", subcore_axis_name="s")

    @functools.partial(
        pl.kernel,
        out_type=jax.ShapeDtypeStruct((TOKENS // 2, 2 * COORDS), jnp.float32),
        mesh=mesh,
        scratch_types=[
            pltpu.VMEM((2, COORDS, TCH), jnp.float32),
            pltpu.VMEM((2, TCH // 2, 2 * COORDS), jnp.float32),
            pltpu.VMEM((COORDS, TAIL), jnp.float32),
            [pltpu.SemaphoreType.DMA] * 2,
            [pltpu.SemaphoreType.DMA] * 2,
        ],
        compiler_params=pltpu.CompilerParams(
            use_tc_tiling_on_sc=True, needs_layout_passes=False
        ),
    )
    def tr_kernel(in_hbm, tail_hbm, out_hbm, stage_v, pack_v, tail_v,
                  isems, osems):
        wid = lax.axis_index("s") * 2 + lax.axis_index("c")
        iota = lax.iota(jnp.int32, 16)
        c0 = wid * 81

        def fire(c, buf):
            t0 = pl.multiple_of(c * TCH, 128)
            pltpu.async_copy(
                in_hbm.at[:, pl.ds(t0, TCH)], stage_v.at[buf], isems[buf]
            )

        def wait_in(c, buf):
            t0 = pl.multiple_of(c * TCH, 128)
            pltpu.make_async_copy(
                in_hbm.at[:, pl.ds(t0, TCH)], stage_v.at[buf], isems[buf]
            ).wait()

        def transpose(buf, width=TCH, src=None):
            sref = stage_v.at[buf] if src is None else src
            pref = pack_v.at[buf]

            @pl.loop(0, 16)
            def _diag(d):
                rot = (iota + d) & 15

                @pl.loop(0, width, step=16)
                def _t(t0):
                    tv = t0 + iota
                    pvec = tv >> 1
                    par = (tv & 1) << 6
                    for k in range(COORDS // 16):
                        ccv = rot + k * 16
                        vec = plsc.load_gather(sref, [ccv, tv])
                        plsc.store_scatter(pref, [pvec, par + ccv], vec)

        def fire_out(c, buf):
            pltpu.async_copy(
                pack_v.at[buf],
                out_hbm.at[pl.ds(c * (TCH // 2), TCH // 2)],
                osems[buf],
            )

        def wait_out(c, buf):
            pltpu.make_async_copy(
                pack_v.at[buf],
                out_hbm.at[pl.ds(c * (TCH // 2), TCH // 2)],
                osems[buf],
            ).wait()

        fire(c0, 0)
        fire(c0 + 1, 1)

        @pl.loop(0, 39)
        def _pair(k):
            for par in range(2):
                j = 2 * k + par
                c = c0 + j
                wait_in(c, par)

                @pl.when(k > 0)
                def _wo():
                    wait_out(c - 2, par)

                transpose(par)
                fire_out(c, par)
                fire(c + 2, par)

        wait_in(c0 + 78, 0)
        wait_out(c0 + 76, 0)
        transpose(0)
        fire_out(c0 + 78, 0)
        fire(c0 + 80, 0)
        wait_in(c0 + 79, 1)
        wait_out(c0 + 77, 1)
        transpose(1)
        fire_out(c0 + 79, 1)
        wait_in(c0 + 80, 0)
        wait_out(c0 + 78, 0)
        transpose(0)
        fire_out(c0 + 80, 0)
        wait_out(c0 + 79, 1)
        wait_out(c0 + 80, 0)

        @pl.when(wid < N_FULL - 32 * 81)
        def _extra():
            c = 32 * 81 + wid
            fire(c, 0)
            wait_in(c, 0)
            transpose(0)
            fire_out(c, 0)
            wait_out(c, 0)

        @pl.when(wid == 31)
        def _tail():
            pltpu.sync_copy(tail_hbm, tail_v)
            transpose(1, width=TAIL, src=tail_v)
            pltpu.sync_copy(
                pack_v.at[1, pl.ds(0, TAIL // 2)],
                out_hbm.at[pl.ds(N_FULL * TCH // 2, TAIL // 2)],
            )

    return tr_kernel(table_ct, table_tail)


def kernel(sequence_bw, vocab_table_tc, pos_table_wc):
    seq_p = sequence_bw.T
    pos_p = pos_table_wc.T
    table2 = _pair_transpose_sc(
        vocab_table_tc.T, vocab_table_tc.T[:, N_FULL * TCH:]
    )
    mesh = plsc.VectorSubcoreMesh(core_axis_name="c", subcore_axis_name="s")

    @functools.partial(
        pl.kernel,
        out_type=jax.ShapeDtypeStruct((WORDS, COORDS, BATCH), jnp.float32),
        mesh=mesh,
        scratch_types=[
            pltpu.VMEM((STAGE_ROWS, BW), jnp.int32),
            pltpu.VMEM((2, BW), jnp.int32),
            pltpu.VMEM((2, BW, GPITCH), jnp.float32),
            pltpu.VMEM((2, COORDS, BW), jnp.float32),
            pltpu.VMEM((COORDS, WORDS), jnp.float32),
            [pltpu.SemaphoreType.DMA] * 2,
            [pltpu.SemaphoreType.DMA] * 2,
        ],
        compiler_params=pltpu.CompilerParams(
            use_tc_tiling_on_sc=True, needs_layout_passes=False
        ),
    )
    def sc_kernel(seq_hbm, table_hbm, pos_hbm, out_hbm,
                  idxs_v, gidx_v, gath_v, ostage_v, pos_v, gsems, ssems):
        wid = lax.axis_index("s") * 2 + lax.axis_index("c")
        w_part = wid // B_COLS
        b0 = pl.multiple_of((wid % B_COLS) * BW, BW)
        w0 = w_part * W_PER_TILE
        w_lo = pl.multiple_of(
            w_part * W_PER_TILE - (w_part * W_PER_TILE) % 8, 8
        )

        pltpu.sync_copy(pos_hbm, pos_v)
        pltpu.sync_copy(
            seq_hbm.at[pl.ds(w_lo, STAGE_ROWS), pl.ds(b0, BW)], idxs_v
        )
        iota = lax.iota(jnp.int32, 16)

        def prep_and_fire(u, buf):
            r = w0 - w_lo + u
            for k in range(BW // 16):
                v = idxs_v[r, pl.ds(k * 16, 16)]
                gidx_v[buf, pl.ds(k * 16, 16)] = v >> 1
            pltpu.async_copy(
                table_hbm.at[gidx_v.at[buf]],
                gath_v.at[buf, :, pl.ds(0, 2 * COORDS)],
                gsems[buf],
            )

        def wait_gather(buf):
            pltpu.make_async_copy(
                table_hbm.at[gidx_v.at[buf]],
                gath_v.at[buf, :, pl.ds(0, 2 * COORDS)],
                gsems[buf],
            ).wait()

        def compute(u, buf):
            r = w0 - w_lo + u
            w_vec = jnp.full((16,), 0, jnp.int32) + (w0 + u)
            h16 = []
            for j in range(BW // 16):
                v = idxs_v[r, pl.ds(j * 16, 16)]
                h16.append((v & 1) << 6)
            pv = [
                plsc.load_gather(pos_v, [k * 16 + iota, w_vec])
                for k in range(COORDS // 16)
            ]
            gref = gath_v.at[buf]
            oref = ostage_v.at[buf]
            @pl.loop(0, 16)
            def _diag(d):
                rot = (iota + d) & 15
                for k in range(COORDS // 16):
                    rotc = rot + (k * 16)
                    posr = pv[k].at[rot].get(mode="promise_in_bounds")
                    for j in range(BW // 16):
                        rows = j * 16 + iota
                        vec = plsc.load_gather(gref, [rows, h16[j] + rotc])
                        plsc.store_scatter(oref, [rotc, rows], vec + posr)

        def fire_store(u, buf):
            pltpu.async_copy(
                ostage_v.at[buf],
                out_hbm.at[w0 + u, :, pl.ds(b0, BW)],
                ssems[buf],
            )

        def wait_store(u, buf):
            pltpu.make_async_copy(
                ostage_v.at[buf],
                out_hbm.at[w0 + u, :, pl.ds(b0, BW)],
                ssems[buf],
            ).wait()

        prep_and_fire(0, 0)
        prep_and_fire(1, 1)

        @pl.loop(0, W_PER_TILE // 2)
        def _pair(k):
            u = 2 * k
            wait_gather(0)

            @pl.when(k > 0)
            def _ws0():
                wait_store(u - 2, 0)

            compute(u, 0)
            fire_store(u, 0)

            @pl.when(k < W_PER_TILE // 2 - 1)
            def _fg0():
                prep_and_fire(u + 2, 0)

            wait_gather(1)

            @pl.when(k > 0)
            def _ws1():
                wait_store(u - 1, 1)

            compute(u + 1, 1)
            fire_store(u + 1, 1)

            @pl.when(k < W_PER_TILE // 2 - 1)
            def _fg1():
                prep_and_fire(u + 3, 1)

        wait_store(W_PER_TILE - 2, 0)
        wait_store(W_PER_TILE - 1, 1)

    out_p = sc_kernel(seq_p, table2, pos_p)
    return out_p.transpose(2, 0, 1)


# speedup vs baseline: 1.2920x; 1.2920x over previous
import functools

import jax
import jax.numpy as jnp
from jax import lax
from jax.experimental import pallas as pl
from jax.experimental.pallas import tpu as pltpu
from jax.experimental.pallas import tpu_sc as plsc

BATCH = 1024
WORDS = 200
COORDS = 64
TOKENS = 1000000
NUM_WORKERS = 32
W_PARTS = 4
B_COLS = 8
W_PER_TILE = WORDS // W_PARTS
BW = 128
STAGE_ROWS = 56
GPITCH = 128


TBLK = 6400


def _pair_transpose_tc(table_ct):
    def body(in_ref, out_ref):
        t = in_ref[...].T
        t3 = t.reshape(TBLK // 2, 2, COORDS)
        out_ref[:, 0:COORDS] = t3[:, 0, :]
        out_ref[:, COORDS:2 * COORDS] = t3[:, 1, :]

    return pl.pallas_call(
        body,
        grid=(pl.cdiv(TOKENS, TBLK),),
        in_specs=[pl.BlockSpec((COORDS, TBLK), lambda i: (0, i))],
        out_specs=pl.BlockSpec((TBLK // 2, 2 * COORDS), lambda i: (i, 0)),
        out_shape=jax.ShapeDtypeStruct((TOKENS // 2, 2 * COORDS), jnp.float32),
    )(table_ct)


def kernel(sequence_bw, vocab_table_tc, pos_table_wc):
    seq_p = sequence_bw.T
    pos_p = pos_table_wc.T
    table2 = _pair_transpose_tc(vocab_table_tc.T)
    mesh = plsc.VectorSubcoreMesh(core_axis_name="c", subcore_axis_name="s")

    @functools.partial(
        pl.kernel,
        out_type=jax.ShapeDtypeStruct((WORDS, COORDS, BATCH), jnp.float32),
        mesh=mesh,
        scratch_types=[
            pltpu.VMEM((STAGE_ROWS, BW), jnp.int32),
            pltpu.VMEM((2, BW), jnp.int32),
            pltpu.VMEM((2, BW, GPITCH), jnp.float32),
            pltpu.VMEM((2, COORDS, BW), jnp.float32),
            pltpu.VMEM((COORDS, WORDS), jnp.float32),
            [pltpu.SemaphoreType.DMA] * 2,
            [pltpu.SemaphoreType.DMA] * 2,
        ],
        compiler_params=pltpu.CompilerParams(
            use_tc_tiling_on_sc=True, needs_layout_passes=False
        ),
    )
    def sc_kernel(seq_hbm, table_hbm, pos_hbm, out_hbm,
                  idxs_v, gidx_v, gath_v, ostage_v, pos_v, gsems, ssems):
        wid = lax.axis_index("s") * 2 + lax.axis_index("c")
        w_part = wid // B_COLS
        b0 = pl.multiple_of((wid % B_COLS) * BW, BW)
        w0 = w_part * W_PER_TILE
        w_lo = pl.multiple_of(
            w_part * W_PER_TILE - (w_part * W_PER_TILE) % 8, 8
        )

        pltpu.sync_copy(pos_hbm, pos_v)
        pltpu.sync_copy(
            seq_hbm.at[pl.ds(w_lo, STAGE_ROWS), pl.ds(b0, BW)], idxs_v
        )
        iota = lax.iota(jnp.int32, 16)

        def prep_and_fire(u, buf):
            r = w0 - w_lo + u
            for k in range(BW // 16):
                v = idxs_v[r, pl.ds(k * 16, 16)]
                gidx_v[buf, pl.ds(k * 16, 16)] = v >> 1
            pltpu.async_copy(
                table_hbm.at[gidx_v.at[buf]],
                gath_v.at[buf, :, pl.ds(0, 2 * COORDS)],
                gsems[buf],
            )

        def wait_gather(buf):
            pltpu.make_async_copy(
                table_hbm.at[gidx_v.at[buf]],
                gath_v.at[buf, :, pl.ds(0, 2 * COORDS)],
                gsems[buf],
            ).wait()

        def compute(u, buf):
            r = w0 - w_lo + u
            w_vec = jnp.full((16,), 0, jnp.int32) + (w0 + u)
            h16 = []
            for j in range(BW // 16):
                v = idxs_v[r, pl.ds(j * 16, 16)]
                h16.append((v & 1) << 6)
            pv = [
                plsc.load_gather(pos_v, [k * 16 + iota, w_vec])
                for k in range(COORDS // 16)
            ]
            gref = gath_v.at[buf]
            oref = ostage_v.at[buf]
            @pl.loop(0, 16)
            def _diag(d):
                rot = (iota + d) & 15
                for k in range(COORDS // 16):
                    rotc = rot + (k * 16)
                    posr = pv[k].at[rot].get(mode="promise_in_bounds")
                    for j in range(BW // 16):
                        rows = j * 16 + iota
                        vec = plsc.load_gather(gref, [rows, h16[j] + rotc])
                        plsc.store_scatter(oref, [rotc, rows], vec + posr)

        def fire_store(u, buf):
            pltpu.async_copy(
                ostage_v.at[buf],
                out_hbm.at[w0 + u, :, pl.ds(b0, BW)],
                ssems[buf],
            )

        def wait_store(u, buf):
            pltpu.make_async_copy(
                ostage_v.at[buf],
                out_hbm.at[w0 + u, :, pl.ds(b0, BW)],
                ssems[buf],
            ).wait()

        prep_and_fire(0, 0)
        prep_and_fire(1, 1)

        @pl.loop(0, W_PER_TILE // 2)
        def _pair(k):
            u = 2 * k
            wait_gather(0)

            @pl.when(k > 0)
            def _ws0():
                wait_store(u - 2, 0)

            compute(u, 0)
            fire_store(u, 0)

            @pl.when(k < W_PER_TILE // 2 - 1)
            def _fg0():
                prep_and_fire(u + 2, 0)

            wait_gather(1)

            @pl.when(k > 0)
            def _ws1():
                wait_store(u - 1, 1)

            compute(u + 1, 1)
            fire_store(u + 1, 1)

            @pl.when(k < W_PER_TILE // 2 - 1)
            def _fg1():
                prep_and_fire(u + 3, 1)

        wait_store(W_PER_TILE - 2, 0)
        wait_store(W_PER_TILE - 1, 1)

    out_p = sc_kernel(seq_p, table2, pos_p)
    return out_p.transpose(2, 0, 1)


# speedup vs baseline: 1.3188x; 1.0208x over previous
import functools

import jax
import jax.numpy as jnp
from jax import lax
from jax.experimental import pallas as pl
from jax.experimental.pallas import tpu as pltpu
from jax.experimental.pallas import tpu_sc as plsc

BATCH = 1024
WORDS = 200
COORDS = 64
TOKENS = 1000000
NUM_WORKERS = 32
W_PARTS = 4
B_COLS = 8
W_PER_TILE = WORDS // W_PARTS
BW = 128
STAGE_ROWS = 56
GPITCH = 128


TBLK = 12800


def _pair_transpose_tc(table_ct):
    def body(in_ref, out_ref):
        t = in_ref[...].T
        t3 = t.reshape(TBLK // 2, 2, COORDS)
        out_ref[:, 0:COORDS] = t3[:, 0, :]
        out_ref[:, COORDS:2 * COORDS] = t3[:, 1, :]

    return pl.pallas_call(
        body,
        grid=(pl.cdiv(TOKENS, TBLK),),
        in_specs=[pl.BlockSpec((COORDS, TBLK), lambda i: (0, i))],
        out_specs=pl.BlockSpec((TBLK // 2, 2 * COORDS), lambda i: (i, 0)),
        out_shape=jax.ShapeDtypeStruct((TOKENS // 2, 2 * COORDS), jnp.float32),
    )(table_ct)


def kernel(sequence_bw, vocab_table_tc, pos_table_wc):
    seq_p = sequence_bw.T
    pos_p = pos_table_wc.T
    table2 = _pair_transpose_tc(vocab_table_tc.T)
    mesh = plsc.VectorSubcoreMesh(core_axis_name="c", subcore_axis_name="s")

    @functools.partial(
        pl.kernel,
        out_type=jax.ShapeDtypeStruct((WORDS, COORDS, BATCH), jnp.float32),
        mesh=mesh,
        scratch_types=[
            pltpu.VMEM((STAGE_ROWS, BW), jnp.int32),
            pltpu.VMEM((2, BW), jnp.int32),
            pltpu.VMEM((2, BW, GPITCH), jnp.float32),
            pltpu.VMEM((2, COORDS, BW), jnp.float32),
            pltpu.VMEM((COORDS, WORDS), jnp.float32),
            [pltpu.SemaphoreType.DMA] * 2,
            [pltpu.SemaphoreType.DMA] * 2,
        ],
        compiler_params=pltpu.CompilerParams(
            use_tc_tiling_on_sc=True, needs_layout_passes=False
        ),
    )
    def sc_kernel(seq_hbm, table_hbm, pos_hbm, out_hbm,
                  idxs_v, gidx_v, gath_v, ostage_v, pos_v, gsems, ssems):
        wid = lax.axis_index("s") * 2 + lax.axis_index("c")
        w_part = wid // B_COLS
        b0 = pl.multiple_of((wid % B_COLS) * BW, BW)
        w0 = w_part * W_PER_TILE
        w_lo = pl.multiple_of(
            w_part * W_PER_TILE - (w_part * W_PER_TILE) % 8, 8
        )

        pltpu.sync_copy(pos_hbm, pos_v)
        pltpu.sync_copy(
            seq_hbm.at[pl.ds(w_lo, STAGE_ROWS), pl.ds(b0, BW)], idxs_v
        )
        iota = lax.iota(jnp.int32, 16)

        def prep_and_fire(u, buf):
            r = w0 - w_lo + u
            for k in range(BW // 16):
                v = idxs_v[r, pl.ds(k * 16, 16)]
                gidx_v[buf, pl.ds(k * 16, 16)] = v >> 1
            pltpu.async_copy(
                table_hbm.at[gidx_v.at[buf]],
                gath_v.at[buf, :, pl.ds(0, 2 * COORDS)],
                gsems[buf],
            )

        def wait_gather(buf):
            pltpu.make_async_copy(
                table_hbm.at[gidx_v.at[buf]],
                gath_v.at[buf, :, pl.ds(0, 2 * COORDS)],
                gsems[buf],
            ).wait()

        def compute(u, buf):
            r = w0 - w_lo + u
            w_vec = jnp.full((16,), 0, jnp.int32) + (w0 + u)
            h16 = []
            for j in range(BW // 16):
                v = idxs_v[r, pl.ds(j * 16, 16)]
                h16.append((v & 1) << 6)
            pv = [
                plsc.load_gather(pos_v, [k * 16 + iota, w_vec])
                for k in range(COORDS // 16)
            ]
            gref = gath_v.at[buf]
            oref = ostage_v.at[buf]
            @pl.loop(0, 16)
            def _diag(d):
                rot = (iota + d) & 15
                for k in range(COORDS // 16):
                    rotc = rot + (k * 16)
                    posr = pv[k].at[rot].get(mode="promise_in_bounds")
                    for j in range(BW // 16):
                        rows = j * 16 + iota
                        vec = plsc.load_gather(gref, [rows, h16[j] + rotc])
                        plsc.store_scatter(oref, [rotc, rows], vec + posr)

        def fire_store(u, buf):
            pltpu.async_copy(
                ostage_v.at[buf],
                out_hbm.at[w0 + u, :, pl.ds(b0, BW)],
                ssems[buf],
            )

        def wait_store(u, buf):
            pltpu.make_async_copy(
                ostage_v.at[buf],
                out_hbm.at[w0 + u, :, pl.ds(b0, BW)],
                ssems[buf],
            ).wait()

        prep_and_fire(0, 0)
        prep_and_fire(1, 1)

        @pl.loop(0, W_PER_TILE // 2)
        def _pair(k):
            u = 2 * k
            wait_gather(0)

            @pl.when(k > 0)
            def _ws0():
                wait_store(u - 2, 0)

            compute(u, 0)
            fire_store(u, 0)

            @pl.when(k < W_PER_TILE // 2 - 1)
            def _fg0():
                prep_and_fire(u + 2, 0)

            wait_gather(1)

            @pl.when(k > 0)
            def _ws1():
                wait_store(u - 1, 1)

            compute(u + 1, 1)
            fire_store(u + 1, 1)

            @pl.when(k < W_PER_TILE // 2 - 1)
            def _fg1():
                prep_and_fire(u + 3, 1)

        wait_store(W_PER_TILE - 2, 0)
        wait_store(W_PER_TILE - 1, 1)

    out_p = sc_kernel(seq_p, table2, pos_p)
    return out_p.transpose(2, 0, 1)
